# baseline (device time: 34744 ns/iter reference)
import jax
import jax.numpy as jnp
from jax import lax
from jax.experimental import pallas as pl
from jax.experimental.pallas import tpu as pltpu

N_DEV = 4
P = 288


def _counts_and_group(x, pos2, c2):
    m, n = x.shape

    def body(x_ref, pos_ref, c_ref, cnt_ref, xg_ref, csend, crecv):
        my = lax.axis_index("i")

        barrier = pltpu.get_barrier_semaphore()
        for k in range(1, N_DEV):
            nbr = lax.rem(my + k, N_DEV)
            pl.semaphore_signal(
                barrier, inc=1,
                device_id=(nbr,), device_id_type=pl.DeviceIdType.MESH,
            )
        pl.semaphore_wait(barrier, N_DEV - 1)

        sends = []
        for k in range(1, N_DEV):
            p = lax.rem(my + k, N_DEV)
            cr = pltpu.make_async_remote_copy(
                src_ref=c_ref,
                dst_ref=cnt_ref.at[pl.ds(my * 8, 8), :],
                send_sem=csend.at[k - 1],
                recv_sem=crecv.at[my],
                device_id=(p,),
                device_id_type=pl.DeviceIdType.MESH,
            )
            cr.start()
            sends.append(cr)

        cnt_ref[pl.ds(my * 8, 8), :] = c_ref[:, :]

        for p in range(N_DEV):
            q = lax.broadcasted_iota(jnp.int32, (P, m), 0) + p * P
            g = (q == pos_ref[0:1, :]).astype(jnp.float32)
            xg_ref[pl.ds(p * P, P), :] = jnp.dot(
                g, x_ref[:, :],
                preferred_element_type=jnp.float32,
                precision=lax.Precision.HIGHEST,
            )

        for cr in sends:
            cr.wait_send()
        for k in range(1, N_DEV):
            r = lax.rem(my + N_DEV - k, N_DEV)
            pltpu.make_async_remote_copy(
                src_ref=c_ref,
                dst_ref=cnt_ref.at[pl.ds(r * 8, 8), :],
                send_sem=csend.at[k - 1],
                recv_sem=crecv.at[r],
                device_id=(r,),
                device_id_type=pl.DeviceIdType.MESH,
            ).wait_recv()

    return pl.pallas_call(
        body,
        out_shape=(
            jax.ShapeDtypeStruct((N_DEV * 8, 128), c2.dtype),
            jax.ShapeDtypeStruct((N_DEV * P, n), x.dtype),
        ),
        in_specs=[
            pl.BlockSpec(memory_space=pltpu.VMEM),
            pl.BlockSpec(memory_space=pltpu.VMEM),
            pl.BlockSpec(memory_space=pltpu.VMEM),
        ],
        out_specs=(
            pl.BlockSpec(memory_space=pltpu.VMEM),
            pl.BlockSpec(memory_space=pltpu.VMEM),
        ),
        scratch_shapes=[
            pltpu.SemaphoreType.DMA((N_DEV - 1,)),
            pltpu.SemaphoreType.DMA((N_DEV,)),
        ],
        compiler_params=pltpu.CompilerParams(collective_id=0),
    )(x, pos2, c2)


def _a2a(xg, o_in):
    mp, n = xg.shape
    m = 1024

    def body(xg_ref, o_ref, out_ref, stg_ref, dsend, drecv):
        my = lax.axis_index("i")

        barrier = pltpu.get_barrier_semaphore()
        for k in range(1, N_DEV):
            nbr = lax.rem(my + k, N_DEV)
            pl.semaphore_signal(
                barrier, inc=1,
                device_id=(nbr,), device_id_type=pl.DeviceIdType.MESH,
            )
        pl.semaphore_wait(barrier, N_DEV - 1)

        sends = []
        for k in range(1, N_DEV):
            p = lax.rem(my + k, N_DEV)
            dr = pltpu.make_async_remote_copy(
                src_ref=xg_ref.at[pl.ds(p * P, P), :],
                dst_ref=stg_ref.at[pl.ds(my * P, P), :],
                send_sem=dsend.at[k - 1],
                recv_sem=drecv.at[my],
                device_id=(p,),
                device_id_type=pl.DeviceIdType.MESH,
            )
            dr.start()
            sends.append(dr)

        zpad = jnp.zeros((m - P, n), jnp.float32)
        own = jnp.concatenate([xg_ref[pl.ds(my * P, P), :], zpad], axis=0)
        out_ref[:, :] = pltpu.roll(own, o_ref[my], 0)

        for dr in sends:
            dr.wait_send()

        for k in range(1, N_DEV):
            r = lax.rem(my + N_DEV - k, N_DEV)
            pltpu.make_async_remote_copy(
                src_ref=xg_ref.at[pl.ds(0, P), :],
                dst_ref=stg_ref.at[pl.ds(r * P, P), :],
                send_sem=dsend.at[k - 1],
                recv_sem=drecv.at[r],
                device_id=(r,),
                device_id_type=pl.DeviceIdType.MESH,
            ).wait_recv()
            seg = jnp.concatenate(
                [stg_ref[pl.ds(r * P, P), :], zpad], axis=0)
            out_ref[:, :] += pltpu.roll(seg, o_ref[r], 0)

    return pl.pallas_call(
        body,
        out_shape=jax.ShapeDtypeStruct((m, n), xg.dtype),
        in_specs=[
            pl.BlockSpec(memory_space=pltpu.VMEM),
            pl.BlockSpec(memory_space=pltpu.SMEM),
        ],
        out_specs=pl.BlockSpec(memory_space=pltpu.VMEM),
        scratch_shapes=[
            pltpu.VMEM((N_DEV * P, n), xg.dtype),
            pltpu.SemaphoreType.DMA((N_DEV - 1,)),
            pltpu.SemaphoreType.DMA((N_DEV,)),
        ],
        compiler_params=pltpu.CompilerParams(collective_id=1),
    )(xg, o_in)


def kernel(x, dest):
    m = x.shape[0]

    dest = dest.astype(jnp.int32)
    tgt = jnp.arange(N_DEV, dtype=jnp.int32)
    masks = (dest[None, :] == tgt[:, None])
    cums = jnp.cumsum(masks.astype(jnp.int32), axis=1)
    rank = jnp.sum(jnp.where(masks, cums - 1, 0), axis=0).astype(jnp.int32)
    pos2 = (dest * P + rank).reshape(1, m)

    cnts = cums[:, -1].astype(jnp.int32)
    c2 = jnp.zeros((8, 128), jnp.int32).at[0, :N_DEV].set(cnts)

    cnt_all, xg = _counts_and_group(x, pos2, c2)

    my = lax.axis_index("i")
    col = jnp.sum(
        cnt_all[::8, :N_DEV]
        * (jnp.arange(N_DEV, dtype=jnp.int32)[None, :] == my),
        axis=1,
    )
    o_in = (jnp.cumsum(col) - col).astype(jnp.int32)

    return _a2a(xg, o_in)


# device time: 31832 ns/iter; 1.0915x vs baseline; 1.0915x over previous
import jax
import jax.numpy as jnp
from jax import lax
from jax.experimental import pallas as pl
from jax.experimental.pallas import tpu as pltpu

N_DEV = 4
P = 288


def _a2a(x, pos2, c2):
    m, n = x.shape

    def slot_dot(pos_ref, x_ref, p):
        q = lax.broadcasted_iota(jnp.int32, (P, m), 0) + p * P
        g = (q == pos_ref[0:1, :]).astype(jnp.float32)
        return jnp.dot(g, x_ref[:, :],
                       preferred_element_type=jnp.float32,
                       precision=lax.Precision.HIGHEST)

    def body(x_ref, pos_ref, c_ref, cnt_ref, stg_ref,
             xg_ref, csend, crecv, dsend, drecv):
        my = lax.axis_index("i")

        barrier = pltpu.get_barrier_semaphore()
        for k in range(1, N_DEV):
            nbr = lax.rem(my + k, N_DEV)
            pl.semaphore_signal(
                barrier, inc=1,
                device_id=(nbr,), device_id_type=pl.DeviceIdType.MESH,
            )
        pl.semaphore_wait(barrier, N_DEV - 1)

        csends = []
        for k in range(1, N_DEV):
            p = lax.rem(my + k, N_DEV)
            cr = pltpu.make_async_remote_copy(
                src_ref=c_ref,
                dst_ref=cnt_ref.at[pl.ds(my * 8, 8), :],
                send_sem=csend.at[k - 1],
                recv_sem=crecv.at[my],
                device_id=(p,),
                device_id_type=pl.DeviceIdType.MESH,
            )
            cr.start()
            csends.append(cr)
        cnt_ref[pl.ds(my * 8, 8), :] = c_ref[:, :]

        dsends = []
        for k in range(1, N_DEV):
            p = lax.rem(my + k, N_DEV)
            xg_ref[pl.ds(p * P, P), :] = slot_dot(pos_ref, x_ref, p)
            dr = pltpu.make_async_remote_copy(
                src_ref=xg_ref.at[pl.ds(p * P, P), :],
                dst_ref=stg_ref.at[pl.ds(my * P, P), :],
                send_sem=dsend.at[k - 1],
                recv_sem=drecv.at[my],
                device_id=(p,),
                device_id_type=pl.DeviceIdType.MESH,
            )
            dr.start()
            dsends.append(dr)

        stg_ref[pl.ds(my * P, P), :] = slot_dot(pos_ref, x_ref, my)

        for cr in csends:
            cr.wait_send()
        for dr in dsends:
            dr.wait_send()

        for k in range(1, N_DEV):
            r = lax.rem(my + N_DEV - k, N_DEV)
            pltpu.make_async_remote_copy(
                src_ref=c_ref,
                dst_ref=cnt_ref.at[pl.ds(r * 8, 8), :],
                send_sem=csend.at[k - 1],
                recv_sem=crecv.at[r],
                device_id=(r,),
                device_id_type=pl.DeviceIdType.MESH,
            ).wait_recv()
            pltpu.make_async_remote_copy(
                src_ref=xg_ref.at[pl.ds(0, P), :],
                dst_ref=stg_ref.at[pl.ds(r * P, P), :],
                send_sem=dsend.at[k - 1],
                recv_sem=drecv.at[r],
                device_id=(r,),
                device_id_type=pl.DeviceIdType.MESH,
            ).wait_recv()

    return pl.pallas_call(
        body,
        out_shape=(
            jax.ShapeDtypeStruct((N_DEV * 8, 128), c2.dtype),
            jax.ShapeDtypeStruct((N_DEV * P, n), x.dtype),
        ),
        in_specs=[
            pl.BlockSpec(memory_space=pltpu.VMEM),
            pl.BlockSpec(memory_space=pltpu.VMEM),
            pl.BlockSpec(memory_space=pltpu.VMEM),
        ],
        out_specs=(
            pl.BlockSpec(memory_space=pltpu.VMEM),
            pl.BlockSpec(memory_space=pltpu.VMEM),
        ),
        scratch_shapes=[
            pltpu.VMEM((N_DEV * P, n), x.dtype),
            pltpu.SemaphoreType.DMA((N_DEV - 1,)),
            pltpu.SemaphoreType.DMA((N_DEV,)),
            pltpu.SemaphoreType.DMA((N_DEV - 1,)),
            pltpu.SemaphoreType.DMA((N_DEV,)),
        ],
        compiler_params=pltpu.CompilerParams(collective_id=0),
    )(x, pos2, c2)


def _compact(stg, o_in):
    mp, n = stg.shape
    m = 1024

    def body(stg_ref, o_ref, out_ref):
        zpad = jnp.zeros((m - P, n), jnp.float32)
        seg0 = jnp.concatenate([stg_ref[pl.ds(0, P), :], zpad], axis=0)
        out_ref[:, :] = pltpu.roll(seg0, o_ref[0], 0)
        for r in range(1, N_DEV):
            seg = jnp.concatenate(
                [stg_ref[pl.ds(r * P, P), :], zpad], axis=0)
            out_ref[:, :] += pltpu.roll(seg, o_ref[r], 0)

    return pl.pallas_call(
        body,
        out_shape=jax.ShapeDtypeStruct((m, n), stg.dtype),
        in_specs=[
            pl.BlockSpec(memory_space=pltpu.VMEM),
            pl.BlockSpec(memory_space=pltpu.SMEM),
        ],
        out_specs=pl.BlockSpec(memory_space=pltpu.VMEM),
    )(stg, o_in)


def kernel(x, dest):
    m = x.shape[0]

    dest = dest.astype(jnp.int32)
    tgt = jnp.arange(N_DEV, dtype=jnp.int32)
    masks = (dest[None, :] == tgt[:, None])
    cums = jnp.cumsum(masks.astype(jnp.int32), axis=1)
    rank = jnp.sum(jnp.where(masks, cums - 1, 0), axis=0).astype(jnp.int32)
    pos2 = (dest * P + rank).reshape(1, m)

    cnts = cums[:, -1].astype(jnp.int32)
    c2 = jnp.zeros((8, 128), jnp.int32).at[0, :N_DEV].set(cnts)

    cnt_all, stg = _a2a(x, pos2, c2)

    my = lax.axis_index("i")
    col = jnp.sum(
        cnt_all[::8, :N_DEV]
        * (jnp.arange(N_DEV, dtype=jnp.int32)[None, :] == my),
        axis=1,
    )
    o_in = (jnp.cumsum(col) - col).astype(jnp.int32)

    return _compact(stg, o_in)


# device time: 30358 ns/iter; 1.1445x vs baseline; 1.0486x over previous
import jax
import jax.numpy as jnp
from jax import lax
from jax.experimental import pallas as pl
from jax.experimental.pallas import tpu as pltpu

N_DEV = 4
P = 288


def _a2a(x, pos2, c2):
    m, n = x.shape

    def slot_dot(pos_ref, x_ref, p):
        q = lax.broadcasted_iota(jnp.int32, (P, m), 0) + p * P
        g = (q == pos_ref[0:1, :]).astype(jnp.float32)
        return jnp.dot(g, x_ref[:, :],
                       preferred_element_type=jnp.float32,
                       precision=lax.Precision.HIGHEST)

    def body(x_ref, pos_ref, c_ref, out_ref,
             xg_ref, stg_ref, cnt_ref, csmem_ref,
             csend, crecv, dsend, drecv, lsem):
        my = lax.axis_index("i")

        barrier = pltpu.get_barrier_semaphore()
        for k in range(1, N_DEV):
            nbr = lax.rem(my + k, N_DEV)
            pl.semaphore_signal(
                barrier, inc=1,
                device_id=(nbr,), device_id_type=pl.DeviceIdType.MESH,
            )
        pl.semaphore_wait(barrier, N_DEV - 1)

        csends = []
        for k in range(1, N_DEV):
            p = lax.rem(my + k, N_DEV)
            cr = pltpu.make_async_remote_copy(
                src_ref=c_ref,
                dst_ref=cnt_ref.at[pl.ds(my * 8, 8), :],
                send_sem=csend.at[k - 1],
                recv_sem=crecv.at[my],
                device_id=(p,),
                device_id_type=pl.DeviceIdType.MESH,
            )
            cr.start()
            csends.append(cr)
        cnt_ref[pl.ds(my * 8, 8), :] = c_ref[:, :]

        dsends = []
        for k in range(1, N_DEV):
            p = lax.rem(my + k, N_DEV)
            xg_ref[pl.ds(p * P, P), :] = slot_dot(pos_ref, x_ref, p)
            dr = pltpu.make_async_remote_copy(
                src_ref=xg_ref.at[pl.ds(p * P, P), :],
                dst_ref=stg_ref.at[pl.ds(my * P, P), :],
                send_sem=dsend.at[k - 1],
                recv_sem=drecv.at[my],
                device_id=(p,),
                device_id_type=pl.DeviceIdType.MESH,
            )
            dr.start()
            dsends.append(dr)

        stg_ref[pl.ds(my * P, P), :] = slot_dot(pos_ref, x_ref, my)

        for cr in csends:
            cr.wait_send()
        for k in range(1, N_DEV):
            r = lax.rem(my + N_DEV - k, N_DEV)
            pltpu.make_async_remote_copy(
                src_ref=c_ref,
                dst_ref=cnt_ref.at[pl.ds(r * 8, 8), :],
                send_sem=csend.at[k - 1],
                recv_sem=crecv.at[r],
                device_id=(r,),
                device_id_type=pl.DeviceIdType.MESH,
            ).wait_recv()
        ccopy = pltpu.make_async_copy(cnt_ref, csmem_ref, lsem)
        ccopy.start()
        ccopy.wait()
        col = [csmem_ref[8 * r, my] for r in range(N_DEV)]

        def off(r):
            o = jnp.int32(0)
            for rp in range(1, N_DEV):
                o += jnp.where(r >= rp, col[rp - 1], 0)
            return o

        zpad = jnp.zeros((m - P, n), jnp.float32)
        own = jnp.concatenate(
            [stg_ref[pl.ds(my * P, P), :], zpad], axis=0)
        out_ref[:, :] = pltpu.roll(own, off(my), 0)

        for dr in dsends:
            dr.wait_send()

        for k in range(1, N_DEV):
            r = lax.rem(my + N_DEV - k, N_DEV)
            pltpu.make_async_remote_copy(
                src_ref=xg_ref.at[pl.ds(0, P), :],
                dst_ref=stg_ref.at[pl.ds(r * P, P), :],
                send_sem=dsend.at[k - 1],
                recv_sem=drecv.at[r],
                device_id=(r,),
                device_id_type=pl.DeviceIdType.MESH,
            ).wait_recv()
            seg = jnp.concatenate(
                [stg_ref[pl.ds(r * P, P), :], zpad], axis=0)
            out_ref[:, :] += pltpu.roll(seg, off(r), 0)

    return pl.pallas_call(
        body,
        out_shape=jax.ShapeDtypeStruct((m, n), x.dtype),
        in_specs=[
            pl.BlockSpec(memory_space=pltpu.VMEM),
            pl.BlockSpec(memory_space=pltpu.VMEM),
            pl.BlockSpec(memory_space=pltpu.VMEM),
        ],
        out_specs=pl.BlockSpec(memory_space=pltpu.VMEM),
        scratch_shapes=[
            pltpu.VMEM((N_DEV * P, n), x.dtype),
            pltpu.VMEM((N_DEV * P, n), x.dtype),
            pltpu.VMEM((N_DEV * 8, 128), jnp.int32),
            pltpu.SMEM((N_DEV * 8, 128), jnp.int32),
            pltpu.SemaphoreType.DMA((N_DEV - 1,)),
            pltpu.SemaphoreType.DMA((N_DEV,)),
            pltpu.SemaphoreType.DMA((N_DEV - 1,)),
            pltpu.SemaphoreType.DMA((N_DEV,)),
            pltpu.SemaphoreType.DMA(()),
        ],
        compiler_params=pltpu.CompilerParams(collective_id=0),
    )(x, pos2, c2)


def kernel(x, dest):
    m = x.shape[0]

    dest = dest.astype(jnp.int32)
    tgt = jnp.arange(N_DEV, dtype=jnp.int32)
    masks = (dest[None, :] == tgt[:, None])
    cums = jnp.cumsum(masks.astype(jnp.int32), axis=1)
    rank = jnp.sum(jnp.where(masks, cums - 1, 0), axis=0).astype(jnp.int32)
    pos2 = (dest * P + rank).reshape(1, m)

    cnts = cums[:, -1].astype(jnp.int32)
    c2 = jnp.zeros((8, 128), jnp.int32).at[0, :N_DEV].set(cnts)

    return _a2a(x, pos2, c2)


# device time: 25440 ns/iter; 1.3657x vs baseline; 1.1933x over previous
import jax
import jax.numpy as jnp
from jax import lax
from jax.experimental import pallas as pl
from jax.experimental.pallas import tpu as pltpu

N_DEV = 4
P = 280


def _a2a(x, d2):
    m, n = x.shape

    def body(x_ref, d_ref, out_ref,
             xg_ref, stg_ref, cscr_ref, cnt_ref, csmem_ref,
             csend, crecv, dsend, drecv, lsem):
        my = lax.axis_index("i")

        barrier = pltpu.get_barrier_semaphore()
        for k in range(1, N_DEV):
            nbr = lax.rem(my + k, N_DEV)
            pl.semaphore_signal(
                barrier, inc=1,
                device_id=(nbr,), device_id_type=pl.DeviceIdType.MESH,
            )
        pl.semaphore_wait(barrier, N_DEV - 1)

        dest = d_ref[:, :]
        tids = lax.broadcasted_iota(jnp.int32, (N_DEV, m), 0)
        masks = (tids == dest).astype(jnp.bfloat16)
        lt = (lax.broadcasted_iota(jnp.int32, (m, m), 0)
              <= lax.broadcasted_iota(jnp.int32, (m, m), 1)
              ).astype(jnp.bfloat16)
        cum = jnp.dot(masks, lt, preferred_element_type=jnp.float32)

        cscr_ref[0:N_DEV, :] = cum[:, m - 128:]
        csends = []
        for k in range(1, N_DEV):
            p = lax.rem(my + k, N_DEV)
            cr = pltpu.make_async_remote_copy(
                src_ref=cscr_ref,
                dst_ref=cnt_ref.at[pl.ds(my * 8, 8), :],
                send_sem=csend.at[k - 1],
                recv_sem=crecv.at[my],
                device_id=(p,),
                device_id_type=pl.DeviceIdType.MESH,
            )
            cr.start()
            csends.append(cr)
        cnt_ref[pl.ds(my * 8, 8), :] = cscr_ref[:, :]

        masks_f = masks.astype(jnp.float32)
        rank = jnp.sum(masks_f * (cum - 1.0), axis=0, keepdims=True)
        pos = dest.astype(jnp.float32) * P + rank

        xf = x_ref[:, :]
        xhi = xf.astype(jnp.bfloat16)
        xmid = (xf - xhi.astype(jnp.float32)).astype(jnp.bfloat16)
        xlo = (xf - xhi.astype(jnp.float32)
               - xmid.astype(jnp.float32)).astype(jnp.bfloat16)

        def slot_rows(p):
            q = (lax.broadcasted_iota(jnp.int32, (P, m), 0) + p * P
                 ).astype(jnp.float32)
            g = (q == pos).astype(jnp.bfloat16)
            acc = jnp.dot(g, xhi, preferred_element_type=jnp.float32)
            acc += jnp.dot(g, xmid, preferred_element_type=jnp.float32)
            acc += jnp.dot(g, xlo, preferred_element_type=jnp.float32)
            return acc

        dsends = []
        for k in range(1, N_DEV):
            p = lax.rem(my + k, N_DEV)
            xg_ref[pl.ds(p * P, P), :] = slot_rows(p)
            dr = pltpu.make_async_remote_copy(
                src_ref=xg_ref.at[pl.ds(p * P, P), :],
                dst_ref=stg_ref.at[pl.ds(my * P, P), :],
                send_sem=dsend.at[k - 1],
                recv_sem=drecv.at[my],
                device_id=(p,),
                device_id_type=pl.DeviceIdType.MESH,
            )
            dr.start()
            dsends.append(dr)

        stg_ref[pl.ds(my * P, P), :] = slot_rows(my)

        for cr in csends:
            cr.wait_send()
        for k in range(1, N_DEV):
            r = lax.rem(my + N_DEV - k, N_DEV)
            pltpu.make_async_remote_copy(
                src_ref=cscr_ref,
                dst_ref=cnt_ref.at[pl.ds(r * 8, 8), :],
                send_sem=csend.at[k - 1],
                recv_sem=crecv.at[r],
                device_id=(r,),
                device_id_type=pl.DeviceIdType.MESH,
            ).wait_recv()
        ccopy = pltpu.make_async_copy(cnt_ref, csmem_ref, lsem)
        ccopy.start()
        ccopy.wait()
        col = [csmem_ref[8 * r + my, 127].astype(jnp.int32)
               for r in range(N_DEV)]

        def off(r):
            o = jnp.int32(0)
            for rp in range(1, N_DEV):
                o += jnp.where(r >= rp, col[rp - 1], 0)
            return o

        zpad = jnp.zeros((m - P, n), jnp.float32)
        own = jnp.concatenate(
            [stg_ref[pl.ds(my * P, P), :], zpad], axis=0)
        out_ref[:, :] = pltpu.roll(own, off(my), 0)

        for dr in dsends:
            dr.wait_send()

        for k in range(1, N_DEV):
            r = lax.rem(my + N_DEV - k, N_DEV)
            pltpu.make_async_remote_copy(
                src_ref=xg_ref.at[pl.ds(0, P), :],
                dst_ref=stg_ref.at[pl.ds(r * P, P), :],
                send_sem=dsend.at[k - 1],
                recv_sem=drecv.at[r],
                device_id=(r,),
                device_id_type=pl.DeviceIdType.MESH,
            ).wait_recv()
            seg = jnp.concatenate(
                [stg_ref[pl.ds(r * P, P), :], zpad], axis=0)
            out_ref[:, :] += pltpu.roll(seg, off(r), 0)

    return pl.pallas_call(
        body,
        out_shape=jax.ShapeDtypeStruct((m, n), x.dtype),
        in_specs=[
            pl.BlockSpec(memory_space=pltpu.VMEM),
            pl.BlockSpec(memory_space=pltpu.VMEM),
        ],
        out_specs=pl.BlockSpec(memory_space=pltpu.VMEM),
        scratch_shapes=[
            pltpu.VMEM((N_DEV * P, n), x.dtype),
            pltpu.VMEM((N_DEV * P, n), x.dtype),
            pltpu.VMEM((8, 128), jnp.float32),
            pltpu.VMEM((N_DEV * 8, 128), jnp.float32),
            pltpu.SMEM((N_DEV * 8, 128), jnp.float32),
            pltpu.SemaphoreType.DMA((N_DEV - 1,)),
            pltpu.SemaphoreType.DMA((N_DEV,)),
            pltpu.SemaphoreType.DMA((N_DEV - 1,)),
            pltpu.SemaphoreType.DMA((N_DEV,)),
            pltpu.SemaphoreType.DMA(()),
        ],
        compiler_params=pltpu.CompilerParams(collective_id=0),
    )(x, d2)


def kernel(x, dest):
    return _a2a(x, dest.astype(jnp.int32).reshape(1, x.shape[0]))
